# baseline (speedup 1.0000x reference)
import functools
import math

import jax
import jax.numpy as jnp
from jax import lax
from jax.experimental import pallas as pl
from jax.experimental.pallas import tpu as pltpu
from jax.experimental.pallas import tpu_sc as plsc

D_MODEL = 64
PAIR_W = 2 * D_MODEL
SCALE = math.sqrt(D_MODEL)
LANES = 16
NBUF = 4
ROW_UNROLL = 2


@functools.lru_cache(maxsize=None)
def _make_emb(R, T):
    info = plsc.get_sparse_core_info()
    nw = info.num_cores * info.num_subcores
    r_per_w = R // nw
    half_t = T // 2
    mesh = plsc.VectorSubcoreMesh(core_axis_name="c", subcore_axis_name="s")

    @functools.partial(
        pl.kernel,
        mesh=mesh,
        out_type=jax.ShapeDtypeStruct((R * T // 2, PAIR_W), jnp.float32),
        scratch_types=[
            pltpu.VMEM((r_per_w, T), jnp.int32),
            pltpu.VMEM((half_t, PAIR_W), jnp.float32),
            *[pltpu.VMEM((T, D_MODEL), jnp.float32) for _ in range(NBUF)],
            *[pltpu.SemaphoreType.DMA for _ in range(NBUF)],
        ],
        compiler_params=pltpu.CompilerParams(use_tc_tiling_on_sc=False),
    )
    def emb(x_hbm, table_hbm, out_hbm, idx_v, obuf, *bufs_sems):
        bufs = bufs_sems[:NBUF]
        sems = bufs_sems[NBUF:]
        wid = lax.axis_index("s") * info.num_cores + lax.axis_index("c")
        r_base = wid * r_per_w

        pltpu.sync_copy(x_hbm.at[pl.ds(r_base, r_per_w)], idx_v)

        for b in range(NBUF):
            pltpu.async_copy(table_hbm.at[idx_v.at[b]], bufs[b], sems[b])

        def group_body(g, carry):
            for b in range(NBUF):
                j = g * NBUF + b
                buf = bufs[b]
                pltpu.make_async_copy(
                    table_hbm.at[idx_v.at[j]], buf, sems[b]
                ).wait()

                def scale_rows(rq, c2):
                    r0 = rq * ROW_UNROLL
                    for rr in range(ROW_UNROLL):
                        for h in range(2):
                            for c in range(D_MODEL // LANES):
                                sl = pl.ds(c * LANES, LANES)
                                obuf[
                                    r0 + rr,
                                    pl.ds(h * D_MODEL + c * LANES, LANES),
                                ] = buf[r0 + rr + h * half_t, sl] * SCALE
                    return c2

                lax.fori_loop(0, half_t // ROW_UNROLL, scale_rows, 0)

                pltpu.sync_copy(
                    obuf, out_hbm.at[pl.ds((r_base + j) * half_t, half_t)]
                )

                @pl.when(j + NBUF < r_per_w)
                def _():
                    pltpu.async_copy(
                        table_hbm.at[idx_v.at[j + NBUF]], buf, sems[b]
                    )

            return carry

        lax.fori_loop(0, r_per_w // NBUF, group_body, 0)

    return emb


def _finish_body(g_ref, out_ref):
    nr = out_ref.shape[0]
    t = out_ref.shape[1]
    half = t // 2
    g = g_ref[...]
    a = g[:, 0:D_MODEL]
    b = g[:, D_MODEL:PAIR_W]
    for i in range(nr):
        out_ref[i, pl.ds(0, half), :] = a[i * half : (i + 1) * half, :]
        out_ref[i, pl.ds(half, half), :] = b[i * half : (i + 1) * half, :]


@functools.lru_cache(maxsize=None)
def _make_finish(R, T, tc_rows=16):
    grid = R // tc_rows
    return pl.pallas_call(
        _finish_body,
        grid=(grid,),
        in_specs=[
            pl.BlockSpec((tc_rows * T // 2, PAIR_W), lambda i: (i, 0)),
        ],
        out_specs=pl.BlockSpec((tc_rows, T, D_MODEL), lambda i: (i, 0, 0)),
        out_shape=jax.ShapeDtypeStruct((R, T, D_MODEL), jnp.float32),
        compiler_params=pltpu.CompilerParams(
            dimension_semantics=("arbitrary",)
        ),
    )


def kernel(x, table):
    R, T = x.shape
    g = _make_emb(R, T)(x.astype(jnp.int32), table)
    return _make_finish(R, T)(g)


# speedup vs baseline: 1.3815x; 1.3815x over previous
import functools
import math

import jax
import jax.numpy as jnp
from jax import lax
from jax.experimental import pallas as pl
from jax.experimental.pallas import tpu as pltpu
from jax.experimental.pallas import tpu_sc as plsc

D_MODEL = 64
SCALE = math.sqrt(D_MODEL)
LANES = 16
NBUF = 4
ROW_UNROLL = 4


@functools.lru_cache(maxsize=None)
def _make_emb(R, T):
    info = plsc.get_sparse_core_info()
    nw = info.num_cores * info.num_subcores
    r_per_w = R // nw
    mesh = plsc.VectorSubcoreMesh(core_axis_name="c", subcore_axis_name="s")

    @functools.partial(
        pl.kernel,
        mesh=mesh,
        out_type=jax.ShapeDtypeStruct((R, T, D_MODEL), jnp.float32),
        scratch_types=[
            pltpu.VMEM((r_per_w, T), jnp.int32),
            *[pltpu.VMEM((T, D_MODEL), jnp.float32) for _ in range(NBUF)],
            *[pltpu.SemaphoreType.DMA for _ in range(NBUF)],
        ],
        compiler_params=pltpu.CompilerParams(use_tc_tiling_on_sc=False),
    )
    def emb(x_hbm, table_hbm, out_hbm, idx_v, *bufs_sems):
        bufs = bufs_sems[:NBUF]
        sems = bufs_sems[NBUF:]
        wid = lax.axis_index("s") * info.num_cores + lax.axis_index("c")
        r_base = wid * r_per_w

        pltpu.sync_copy(x_hbm.at[pl.ds(r_base, r_per_w)], idx_v)

        for b in range(NBUF):
            pltpu.async_copy(table_hbm.at[idx_v.at[b]], bufs[b], sems[b])

        def group_body(g, carry):
            for b in range(NBUF):
                j = g * NBUF + b
                buf = bufs[b]
                pltpu.make_async_copy(
                    table_hbm.at[idx_v.at[j]], buf, sems[b]
                ).wait()

                def scale_rows(rq, c2):
                    r0 = rq * ROW_UNROLL
                    for rr in range(ROW_UNROLL):
                        for c in range(D_MODEL // LANES):
                            sl = pl.ds(c * LANES, LANES)
                            buf[r0 + rr, sl] = buf[r0 + rr, sl] * SCALE
                    return c2

                lax.fori_loop(0, T // ROW_UNROLL, scale_rows, 0)

                pltpu.sync_copy(buf, out_hbm.at[r_base + j])

                @pl.when(j + NBUF < r_per_w)
                def _():
                    pltpu.async_copy(table_hbm.at[idx_v.at[j + NBUF]], buf, sems[b])

            return carry

        lax.fori_loop(0, r_per_w // NBUF, group_body, 0)

    return emb


def kernel(x, table):
    return _make_emb(x.shape[0], x.shape[1])(x.astype(jnp.int32), table)
